# baseline (speedup 1.0000x reference)
import functools

import jax
import jax.numpy as jnp
from jax import lax
from jax.experimental import pallas as pl
from jax.experimental.pallas import tpu as pltpu
from jax.experimental.pallas import tpu_sc as plsc

_CODES = 1024
_DIM = 256
_PIX = 8 * 32 * 32
_BP = 1024


def _argmin_body(z_ref, w_ref, idx_ref):
    z = jnp.transpose(z_ref[0], (1, 0))
    w = w_ref[...]
    zsq = jnp.sum(z * z, axis=1, keepdims=True)
    wsq = jnp.sum(w * w, axis=1)
    mm = lax.dot_general(z, w, (((1,), (1,)), ((), ())),
                         preferred_element_type=jnp.float32)
    d = (zsq + wsq[None, :]) - 2.0 * mm
    dmin = jnp.min(d, axis=1, keepdims=True)
    ii = lax.broadcasted_iota(jnp.int32, d.shape, 1)
    idx = jnp.min(jnp.where(d == dmin, ii, jnp.int32(_CODES)), axis=1)
    idx_ref[...] = idx[None, None, :]


def _tc_argmin(z3, W):
    nblk = _PIX // _BP
    return pl.pallas_call(
        _argmin_body,
        grid=(nblk,),
        in_specs=[
            pl.BlockSpec((1, _DIM, _BP), lambda i: (i, 0, 0)),
            pl.BlockSpec((_CODES, _DIM), lambda i: (0, 0)),
        ],
        out_specs=pl.BlockSpec((1, 1, _BP), lambda i: (i, 0, 0)),
        out_shape=jax.ShapeDtypeStruct((nblk, 1, _BP), jnp.int32),
    )(z3, W)


_NW = 32
_BPW = _PIX // _NW


@functools.cache
def _sc_gather_fn():
    @functools.partial(
        pl.kernel,
        out_type=jax.ShapeDtypeStruct((_PIX, _DIM), jnp.float32),
        mesh=plsc.VectorSubcoreMesh(core_axis_name="c", subcore_axis_name="s"),
        scratch_types=[
            pltpu.VMEM((_BPW,), jnp.int32),
            pltpu.VMEM((_BPW, _DIM), jnp.float32),
            pltpu.SemaphoreType.DMA,
        ],
    )
    def _sc_gather(table_hbm, idx_hbm, out_hbm, idx_v, rows_v, sem):
        wid = lax.axis_index("s") * 2 + lax.axis_index("c")
        base = wid * _BPW
        pltpu.sync_copy(idx_hbm.at[pl.ds(base, _BPW)], idx_v)
        pltpu.async_copy(table_hbm.at[idx_v], rows_v, sem).wait()
        pltpu.sync_copy(rows_v, out_hbm.at[pl.ds(base, _BPW)])

    return _sc_gather


def kernel(z, W):
    z3 = z.reshape(8, _DIM, _BP)
    idx = _tc_argmin(z3, W).reshape(_PIX)
    zq_flat = _sc_gather_fn()(W, idx)
    z_q = zq_flat.reshape(8, 32, 32, _DIM)
    return jnp.transpose(z_q, (0, 3, 1, 2))


# speedup vs baseline: 1.2426x; 1.2426x over previous
import functools

import jax
import jax.numpy as jnp
from jax import lax
from jax.experimental import pallas as pl
from jax.experimental.pallas import tpu as pltpu
from jax.experimental.pallas import tpu_sc as plsc

_CODES = 1024
_DIM = 256
_PIX = 8 * 32 * 32
_BP = 1024


def _argmin_body(z_ref, w_ref, idx_ref):
    z = z_ref[...]
    w = w_ref[...]
    zsq = jnp.sum(z * z, axis=1, keepdims=True)
    wsq = jnp.sum(w * w, axis=1)
    mm = lax.dot_general(z, w, (((1,), (1,)), ((), ())),
                         preferred_element_type=jnp.float32)
    d = (zsq + wsq[None, :]) - 2.0 * mm
    idx = jnp.argmin(d, axis=1).astype(jnp.int32)
    idx_ref[...] = idx[None, None, :]


def _tc_argmin(z_flat, W):
    nblk = _PIX // _BP
    return pl.pallas_call(
        _argmin_body,
        grid=(nblk,),
        in_specs=[
            pl.BlockSpec((_BP, _DIM), lambda i: (i, 0)),
            pl.BlockSpec((_CODES, _DIM), lambda i: (0, 0)),
        ],
        out_specs=pl.BlockSpec((1, 1, _BP), lambda i: (i, 0, 0)),
        out_shape=jax.ShapeDtypeStruct((nblk, 1, _BP), jnp.int32),
    )(z_flat, W)


_NW = 32
_BPW = _PIX // _NW


@functools.cache
def _sc_gather_fn():
    @functools.partial(
        pl.kernel,
        out_type=jax.ShapeDtypeStruct((_PIX, _DIM), jnp.float32),
        mesh=plsc.VectorSubcoreMesh(core_axis_name="c", subcore_axis_name="s"),
        scratch_types=[
            pltpu.VMEM((_BPW,), jnp.int32),
            pltpu.VMEM((_BPW, _DIM), jnp.float32),
            pltpu.SemaphoreType.DMA,
        ],
    )
    def _sc_gather(table_hbm, idx_hbm, out_hbm, idx_v, rows_v, sem):
        wid = lax.axis_index("s") * 2 + lax.axis_index("c")
        base = wid * _BPW
        pltpu.sync_copy(idx_hbm.at[pl.ds(base, _BPW)], idx_v)
        pltpu.async_copy(table_hbm.at[idx_v], rows_v, sem).wait()
        pltpu.sync_copy(rows_v, out_hbm.at[pl.ds(base, _BPW)])

    return _sc_gather


def kernel(z, W):
    zp = jnp.transpose(z, (0, 2, 3, 1))
    z_flat = zp.reshape(_PIX, _DIM)
    idx = _tc_argmin(z_flat, W).reshape(_PIX)
    zq_flat = _sc_gather_fn()(W, idx)
    z_q = zq_flat.reshape(8, 32, 32, _DIM)
    return jnp.transpose(z_q, (0, 3, 1, 2))
